# Initial kernel scaffold; baseline (speedup 1.0000x reference)
import jax
import jax.numpy as jnp
from jax.experimental import pallas as pl


def kernel(embedding, edge_index):
    raise NotImplementedError("write your pallas kernel here")


# speedup vs baseline: 2.2796x; 2.2796x over previous
import functools

import jax
import jax.numpy as jnp
from jax import lax
from jax.experimental import pallas as pl
from jax.experimental.pallas import tpu as pltpu
from jax.experimental.pallas import tpu_sc as plsc

_NC = 2
_NS = 16
_NW = _NC * _NS
_L = 16
_C = 64
_D = 256


@functools.lru_cache(maxsize=None)
def _make_kernel(E):
    assert E % _C == 0
    n_chunks = E // _C
    nt = -(-n_chunks // _NW)
    mesh = plsc.VectorSubcoreMesh(core_axis_name="c", subcore_axis_name="s")

    @functools.partial(
        pl.kernel,
        out_type=jax.ShapeDtypeStruct((E,), jnp.float32),
        mesh=mesh,
        compiler_params=pltpu.CompilerParams(needs_layout_passes=False),
        scratch_types=[
            pltpu.VMEM((_C,), jnp.int32),
            pltpu.VMEM((_C,), jnp.int32),
            pltpu.VMEM((_C, _D), jnp.float32),
            pltpu.VMEM((_C, _D), jnp.float32),
            pltpu.VMEM((_C,), jnp.float32),
            pltpu.VMEM((_L * _L,), jnp.float32),
            pltpu.SemaphoreType.DMA,
        ],
    )
    def ker(emb, src, dst, out, sidx, didx, srows, drows, scores, accbuf, sem):
        wid = lax.axis_index("s") * _NC + lax.axis_index("c")

        @pl.loop(0, nt)
        def _chunks(t):
            cid = t * _NW + wid

            @pl.when(cid < n_chunks)
            def _():
                off = cid * _C
                pltpu.sync_copy(src.at[pl.ds(off, _C)], sidx)
                pltpu.sync_copy(dst.at[pl.ds(off, _C)], didx)
                cp1 = pltpu.async_copy(emb.at[sidx], srows, sem)
                cp2 = pltpu.async_copy(emb.at[didx], drows, sem)
                cp1.wait()
                cp2.wait()

                @pl.loop(0, _C // _L)
                def _groups(j):
                    for m in range(_L):
                        e = j * _L + m
                        acc = srows[e, pl.ds(0, _L)] * drows[e, pl.ds(0, _L)]
                        for k in range(1, _D // _L):
                            acc = acc + (srows[e, pl.ds(k * _L, _L)]
                                         * drows[e, pl.ds(k * _L, _L)])
                        accbuf[pl.ds(m * _L, _L)] = acc
                    iot = lax.iota(jnp.int32, _L) * _L
                    svec = plsc.load_gather(accbuf, [iot])
                    for l in range(1, _L):
                        svec = svec + plsc.load_gather(accbuf, [iot + l])
                    scores[pl.ds(j * _L, _L)] = svec

                pltpu.sync_copy(scores, out.at[pl.ds(off, _C)])

    return ker


def kernel(embedding, edge_index):
    E = edge_index.shape[1]
    ei = edge_index.astype(jnp.int32)
    out = _make_kernel(E)(embedding, ei[0], ei[1])
    return out[:, None]
